# Initial kernel scaffold; baseline (speedup 1.0000x reference)
import jax
import jax.numpy as jnp
from jax.experimental import pallas as pl


def kernel(x, xyz, x1, xyz1, Wq, bq, Wk, bk, Wv, bv, pA_W, pA_b, p_ln_g, p_ln_b, pB_W, pB_b, qA_W, qA_b, q_ln_g, q_ln_b, qB_W, qB_b, w_ln1_g, w_ln1_b, wA_W, wA_b, w_ln2_g, w_ln2_b, wB_W, wB_b):
    raise NotImplementedError("write your pallas kernel here")


# speedup vs baseline: 6.0475x; 6.0475x over previous
import functools

import jax
import jax.numpy as jnp
from jax import lax
from jax.experimental import pallas as pl
from jax.experimental.pallas import tpu as pltpu
from jax.experimental.pallas import tpu_sc as plsc

_B, _N1, _NP, _NS = 4, 8192, 512, 16
_IN_P, _OUT_P, _MID, _SHARE = 64, 128, 64, 8
_R2 = 0.2 ** 2
_EPS = 1e-5
_HI = jax.lax.Precision.HIGHEST

_SBLK = 64
_CB = 128
_NBQ = _NP // _SBLK
_NCB = _NP // _CB


def _fps_body(xyz_ref, fps_ref, nxyz_ref):
    X = xyz_ref[0, 0]
    Y = xyz_ref[0, 1]
    Z = xyz_ref[0, 2]
    I = (lax.broadcasted_iota(jnp.int32, (64, 128), 0) * 128
         + lax.broadcasted_iota(jnp.int32, (64, 128), 1))
    lane = lax.broadcasted_iota(jnp.int32, (1, 128), 1)

    def step(k, carry):
        D, f = carry
        row = f // 128
        col = f - row * 128
        sel = lane == col
        xr = xyz_ref[0, 0, pl.ds(row, 1), :]
        yr = xyz_ref[0, 1, pl.ds(row, 1), :]
        zr = xyz_ref[0, 2, pl.ds(row, 1), :]
        cx = jnp.sum(jnp.where(sel, xr, 0.0))
        cy = jnp.sum(jnp.where(sel, yr, 0.0))
        cz = jnp.sum(jnp.where(sel, zr, 0.0))
        dist = (X - cx) ** 2 + (Y - cy) ** 2 + (Z - cz) ** 2
        D = jnp.minimum(D, dist)
        m = jnp.max(D)
        f_next = jnp.min(jnp.where(D == m, I, jnp.int32(_N1)))
        fps_ref[pl.ds(k, 1), :] = jnp.full((1, 1), f, jnp.int32)
        nxyz_ref[pl.ds(k, 1), 0:1] = jnp.full((1, 1), cx)
        nxyz_ref[pl.ds(k, 1), 1:2] = jnp.full((1, 1), cy)
        nxyz_ref[pl.ds(k, 1), 2:3] = jnp.full((1, 1), cz)
        return D, f_next

    D0 = jnp.full((64, 128), 1e10, jnp.float32)
    lax.fori_loop(0, _NP, step, (D0, jnp.int32(0)))


def _run_fps(xyzr):
    return pl.pallas_call(
        _fps_body,
        grid=(_B,),
        in_specs=[pl.BlockSpec((1, 3, 64, 128), lambda b: (b, 0, 0, 0))],
        out_specs=[
            pl.BlockSpec((_NP, 1), lambda b: (b, 0)),
            pl.BlockSpec((_NP, 3), lambda b: (b, 0)),
        ],
        out_shape=[
            jax.ShapeDtypeStruct((_B * _NP, 1), jnp.int32),
            jax.ShapeDtypeStruct((_B * _NP, 3), jnp.float32),
        ],
        compiler_params=pltpu.CompilerParams(
            dimension_semantics=("arbitrary",)),
    )(xyzr)


def _bq_body(xyzt_ref, nxyz_ref, idx_ref):
    b = pl.program_id(0)
    px = xyzt_ref[0, 0:1, :]
    py = xyzt_ref[0, 1:2, :]
    pz = xyzt_ref[0, 2:3, :]
    pn = px * px + py * py + pz * pz
    nx = nxyz_ref[:, 0:1]
    ny = nxyz_ref[:, 1:2]
    nz = nxyz_ref[:, 2:3]
    cn = nx * nx + ny * ny + nz * nz
    dot = jnp.dot(nxyz_ref[...], xyzt_ref[0],
                  preferred_element_type=jnp.float32)
    sqd = cn + pn - 2.0 * dot
    iotaL = lax.broadcasted_iota(jnp.int32, (_SBLK, _N1), 1)
    cand0 = jnp.where(sqd <= _R2, iotaL, jnp.int32(_N1))
    slot = lax.broadcasted_iota(jnp.int32, (_SBLK, _NS), 1)

    def step(t, carry):
        cand, acc = carry
        m = jnp.min(cand, axis=1, keepdims=True)
        acc = jnp.where(slot == t, m, acc)
        cand = jnp.where(cand == m, jnp.int32(_N1), cand)
        return cand, acc

    _, acc = lax.fori_loop(
        0, _NS, step, (cand0, jnp.zeros((_SBLK, _NS), jnp.int32)))
    first = acc[:, 0:1]
    acc = jnp.where(acc == _N1, first, acc)
    idx_ref[...] = acc + b * _N1


def _run_bq(xyzt, nxyz):
    return pl.pallas_call(
        _bq_body,
        grid=(_B, _NBQ),
        in_specs=[
            pl.BlockSpec((1, 3, _N1), lambda b, c: (b, 0, 0)),
            pl.BlockSpec((_SBLK, 3), lambda b, c: (b * _NBQ + c, 0)),
        ],
        out_specs=pl.BlockSpec((_SBLK, _NS), lambda b, c: (b * _NBQ + c, 0)),
        out_shape=jax.ShapeDtypeStruct((_B * _NP, _NS), jnp.int32),
        compiler_params=pltpu.CompilerParams(
            dimension_semantics=("arbitrary", "arbitrary")),
    )(xyzt, nxyz)


def _sc_gather(table, idx):
    n, D = idx.shape[0], table.shape[1]
    info = plsc.get_sparse_core_info()
    nw = info.num_cores * info.num_subcores
    b_per_w = n // nw
    nchunk = 2
    chunk = b_per_w // nchunk
    mesh = plsc.VectorSubcoreMesh(core_axis_name="c", subcore_axis_name="s")

    @functools.partial(
        pl.kernel, mesh=mesh,
        out_type=jax.ShapeDtypeStruct((n, D), jnp.float32),
        scratch_types=[
            pltpu.VMEM((chunk,), jnp.int32),
            pltpu.VMEM((chunk, D), jnp.float32),
            pltpu.SemaphoreType.DMA,
        ],
    )
    def k(table_hbm, idx_hbm, out_hbm, idx_v, rows_v, sem):
        wid = lax.axis_index("s") * info.num_cores + lax.axis_index("c")
        for j in range(nchunk):
            base = wid * b_per_w + j * chunk
            pltpu.sync_copy(idx_hbm.at[pl.ds(base, chunk)], idx_v)
            pltpu.async_copy(table_hbm.at[idx_v], rows_v, sem).wait()
            pltpu.sync_copy(rows_v, out_hbm.at[pl.ds(base, chunk)])

    return k(table, idx)


def _ln(h, g, b):
    m = jnp.mean(h, axis=-1, keepdims=True)
    v = jnp.mean((h - m) ** 2, axis=-1, keepdims=True)
    return (h - m) / jnp.sqrt(v + _EPS) * g + b


def _dense_body(g_ref, nrep_ref, xT_ref, xyzT_ref,
                Wq_ref, bq_ref, Wk_ref, bk_ref, Wv_ref, bv_ref,
                pA_ref, pAb_ref, plg_ref, plb_ref, pB_ref, pBb_ref,
                qA_ref, qAb_ref, qlg_ref, qlb_ref, qB_ref, qBb_ref,
                l1g_ref, l1b_ref, wA_ref, wAb_ref,
                l2g_ref, l2b_ref, wB_ref, wBb_ref,
                wlog_ref, xv_ref):
    g = g_ref[0]
    gx1 = g[:, 0:_IN_P]
    gxyz = g[:, _IN_P:_IN_P + 3]
    grel = gxyz - nrep_ref[0]

    x_k = jnp.dot(gx1, Wk_ref[...],
                  preferred_element_type=jnp.float32) + bk_ref[...]
    x_v = jnp.dot(gx1, Wv_ref[...],
                  preferred_element_type=jnp.float32) + bv_ref[...]

    h = jnp.dot(grel, pA_ref[...],
                preferred_element_type=jnp.float32) + pAb_ref[...]
    h = jax.nn.relu(_ln(h, plg_ref[...], plb_ref[...]))
    p_r = jnp.dot(h, pB_ref[...],
                  preferred_element_type=jnp.float32) + pBb_ref[...]
    pr_red = p_r[:, 0:_MID] + p_r[:, _MID:_OUT_P]

    a = xyzT_ref[0]
    h1 = jnp.dot(a, qA_ref[...],
                 preferred_element_type=jnp.float32) + qAb_ref[...]
    h1 = jax.nn.relu(_ln(h1, qlg_ref[...], qlb_ref[...]))
    p_r_1 = jnp.dot(h1, qB_ref[...],
                    preferred_element_type=jnp.float32) + qBb_ref[...]
    x_q = jnp.dot(xT_ref[0], Wq_ref[...],
                  preferred_element_type=jnp.float32) + bq_ref[...]
    q = x_q + p_r_1
    q_rep = jnp.broadcast_to(q[:, None, :], (_CB, _NS, _MID)).reshape(
        _CB * _NS, _MID)

    r_qk = x_k - q_rep + pr_red
    hw = jax.nn.relu(_ln(r_qk, l1g_ref[...], l1b_ref[...]))
    hw = jnp.dot(hw, wA_ref[...],
                 preferred_element_type=jnp.float32) + wAb_ref[...]
    hw = jax.nn.relu(_ln(hw, l2g_ref[...], l2b_ref[...]))
    wlog = jnp.dot(hw, wB_ref[...],
                   preferred_element_type=jnp.float32) + wBb_ref[...]

    wlog_ref[0] = wlog
    xv_ref[0] = x_v + p_r


def _run_dense(g, nrep, xT, xyzT, weights):
    nblk = _B * _NCB
    wspecs = [pl.BlockSpec(w.shape, functools.partial(
        lambda nd, b, c: (0,) * nd, w.ndim)) for w in weights]
    return pl.pallas_call(
        _dense_body,
        grid=(_B, _NCB),
        in_specs=[
            pl.BlockSpec((1, _CB * _NS, 128),
                         lambda b, c: (b * _NCB + c, 0, 0)),
            pl.BlockSpec((1, _CB * _NS, 3),
                         lambda b, c: (b * _NCB + c, 0, 0)),
            pl.BlockSpec((1, _CB, _IN_P), lambda b, c: (b * _NCB + c, 0, 0)),
            pl.BlockSpec((1, _CB, 3), lambda b, c: (b * _NCB + c, 0, 0)),
        ] + wspecs,
        out_specs=[
            pl.BlockSpec((1, _CB * _NS, _NS),
                         lambda b, c: (b * _NCB + c, 0, 0)),
            pl.BlockSpec((1, _CB * _NS, _OUT_P),
                         lambda b, c: (b * _NCB + c, 0, 0)),
        ],
        out_shape=[
            jax.ShapeDtypeStruct((nblk, _CB * _NS, _NS), jnp.float32),
            jax.ShapeDtypeStruct((nblk, _CB * _NS, _OUT_P), jnp.float32),
        ],
        compiler_params=pltpu.CompilerParams(
            dimension_semantics=("arbitrary", "arbitrary")),
    )(g, nrep, xT, xyzT, *weights)


def _agg_body(wlog_ref, xv_ref, out_ref):
    wlog = wlog_ref[0]
    mx = jnp.max(wlog, axis=0, keepdims=True)
    e = jnp.exp(wlog - mx)
    w = e / jnp.sum(e, axis=0, keepdims=True)
    acc = jnp.zeros((_NP, _OUT_P), jnp.float32)
    for t in range(_NS):
        wt = w[:, t * _NS:(t + 1) * _NS]
        wt8 = jnp.concatenate([wt] * _SHARE, axis=1)
        acc = acc + xv_ref[0, :, t, :] * wt8
    out_ref[0] = acc


def _run_agg(wlog, xv):
    return pl.pallas_call(
        _agg_body,
        grid=(_B,),
        in_specs=[
            pl.BlockSpec((1, _NP, _NS * _NS), lambda b: (b, 0, 0)),
            pl.BlockSpec((1, _NP, _NS, _OUT_P), lambda b: (b, 0, 0, 0)),
        ],
        out_specs=pl.BlockSpec((1, _NP, _OUT_P), lambda b: (b, 0, 0)),
        out_shape=jax.ShapeDtypeStruct((_B, _NP, _OUT_P), jnp.float32),
        compiler_params=pltpu.CompilerParams(
            dimension_semantics=("arbitrary",)),
    )(wlog, xv)


def kernel(x, xyz, x1, xyz1, Wq, bq, Wk, bk, Wv, bv, pA_W, pA_b, p_ln_g,
           p_ln_b, pB_W, pB_b, qA_W, qA_b, q_ln_g, q_ln_b, qB_W, qB_b,
           w_ln1_g, w_ln1_b, wA_W, wA_b, w_ln2_g, w_ln2_b, wB_W, wB_b):
    xyzt = jnp.transpose(xyz1, (0, 2, 1))
    xyzr = xyzt.reshape(_B, 3, 64, 128)
    fps_flat, nxyz_flat = _run_fps(xyzr)
    new_xyz = nxyz_flat.reshape(_B, _NP, 3)

    idx_g = _run_bq(xyzt, nxyz_flat)

    table = jnp.concatenate(
        [x1, xyz1, jnp.zeros((_B, _N1, 61), jnp.float32)],
        axis=-1).reshape(_B * _N1, 128)
    g = _sc_gather(table, idx_g.reshape(-1))

    nrep = jnp.broadcast_to(
        new_xyz[:, :, None, :], (_B, _NP, _NS, 3))
    g4 = g.reshape(_B * _NCB, _CB * _NS, 128)
    nrep4 = nrep.reshape(_B * _NCB, _CB * _NS, 3)
    xT = jnp.transpose(x, (0, 2, 1)).reshape(_B * _NCB, _CB, _IN_P)
    xyzT = jnp.transpose(xyz, (0, 2, 1)).reshape(_B * _NCB, _CB, 3)

    weights = [
        Wq, bq.reshape(1, -1), Wk, bk.reshape(1, -1), Wv, bv.reshape(1, -1),
        pA_W, pA_b.reshape(1, -1), p_ln_g.reshape(1, -1),
        p_ln_b.reshape(1, -1), pB_W, pB_b.reshape(1, -1),
        qA_W, qA_b.reshape(1, -1), q_ln_g.reshape(1, -1),
        q_ln_b.reshape(1, -1), qB_W, qB_b.reshape(1, -1),
        w_ln1_g.reshape(1, -1), w_ln1_b.reshape(1, -1), wA_W,
        wA_b.reshape(1, -1), w_ln2_g.reshape(1, -1), w_ln2_b.reshape(1, -1),
        wB_W, wB_b.reshape(1, -1),
    ]
    wlog, xv = _run_dense(g4, nrep4, xT, xyzT, weights)

    wlog2 = wlog.reshape(_B, _NP, _NS * _NS)
    xv4 = xv.reshape(_B, _NP, _NS, _OUT_P)
    out = _run_agg(wlog2, xv4)
    return out, new_xyz


# speedup vs baseline: 6.0478x; 1.0000x over previous
import functools

import jax
import jax.numpy as jnp
from jax import lax
from jax.experimental import pallas as pl
from jax.experimental.pallas import tpu as pltpu
from jax.experimental.pallas import tpu_sc as plsc

_B, _N1, _NP, _NS = 4, 8192, 512, 16
_IN_P, _OUT_P, _MID, _SHARE = 64, 128, 64, 8
_R2 = 0.2 ** 2
_EPS = 1e-5
_HI = jax.lax.Precision.HIGHEST

_SBLK = 64
_CB = 128
_NBQ = _NP // _SBLK
_NCB = _NP // _CB


def _fps_body(xyz_ref, fps_ref, nxyz_ref):
    X = xyz_ref[0, 0]
    Y = xyz_ref[0, 1]
    Z = xyz_ref[0, 2]
    I = (lax.broadcasted_iota(jnp.int32, (64, 128), 0) * 128
         + lax.broadcasted_iota(jnp.int32, (64, 128), 1))
    lane = lax.broadcasted_iota(jnp.int32, (1, 128), 1)

    def step(k, carry):
        D, f = carry
        row = f // 128
        col = f - row * 128
        sel = lane == col
        xr = xyz_ref[0, 0, pl.ds(row, 1), :]
        yr = xyz_ref[0, 1, pl.ds(row, 1), :]
        zr = xyz_ref[0, 2, pl.ds(row, 1), :]
        cx = jnp.sum(jnp.where(sel, xr, 0.0))
        cy = jnp.sum(jnp.where(sel, yr, 0.0))
        cz = jnp.sum(jnp.where(sel, zr, 0.0))
        dist = (X - cx) ** 2 + (Y - cy) ** 2 + (Z - cz) ** 2
        D = jnp.minimum(D, dist)
        m = jnp.max(D)
        f_next = jnp.min(jnp.where(D == m, I, jnp.int32(_N1)))
        fps_ref[pl.ds(k, 1), :] = jnp.full((1, 1), f, jnp.int32)
        nxyz_ref[pl.ds(k, 1), 0:1] = jnp.full((1, 1), cx)
        nxyz_ref[pl.ds(k, 1), 1:2] = jnp.full((1, 1), cy)
        nxyz_ref[pl.ds(k, 1), 2:3] = jnp.full((1, 1), cz)
        return D, f_next

    D0 = jnp.full((64, 128), 1e10, jnp.float32)
    lax.fori_loop(0, _NP, step, (D0, jnp.int32(0)))


def _run_fps(xyzr):
    return pl.pallas_call(
        _fps_body,
        grid=(_B,),
        in_specs=[pl.BlockSpec((1, 3, 64, 128), lambda b: (b, 0, 0, 0))],
        out_specs=[
            pl.BlockSpec((_NP, 1), lambda b: (b, 0)),
            pl.BlockSpec((_NP, 3), lambda b: (b, 0)),
        ],
        out_shape=[
            jax.ShapeDtypeStruct((_B * _NP, 1), jnp.int32),
            jax.ShapeDtypeStruct((_B * _NP, 3), jnp.float32),
        ],
        compiler_params=pltpu.CompilerParams(
            dimension_semantics=("parallel",)),
    )(xyzr)


def _bq_body(xyzt_ref, nxyz_ref, idx_ref):
    b = pl.program_id(0)
    px = xyzt_ref[0, 0:1, :]
    py = xyzt_ref[0, 1:2, :]
    pz = xyzt_ref[0, 2:3, :]
    pn = px * px + py * py + pz * pz
    nx = nxyz_ref[:, 0:1]
    ny = nxyz_ref[:, 1:2]
    nz = nxyz_ref[:, 2:3]
    cn = nx * nx + ny * ny + nz * nz
    dot = jnp.dot(nxyz_ref[...], xyzt_ref[0],
                  preferred_element_type=jnp.float32)
    sqd = cn + pn - 2.0 * dot
    iotaL = lax.broadcasted_iota(jnp.int32, (_SBLK, _N1), 1)
    cand0 = jnp.where(sqd <= _R2, iotaL, jnp.int32(_N1))
    slot = lax.broadcasted_iota(jnp.int32, (_SBLK, _NS), 1)

    def step(t, carry):
        cand, acc = carry
        m = jnp.min(cand, axis=1, keepdims=True)
        acc = jnp.where(slot == t, m, acc)
        cand = jnp.where(cand == m, jnp.int32(_N1), cand)
        return cand, acc

    _, acc = lax.fori_loop(
        0, _NS, step, (cand0, jnp.zeros((_SBLK, _NS), jnp.int32)))
    first = acc[:, 0:1]
    acc = jnp.where(acc == _N1, first, acc)
    idx_ref[...] = acc + b * _N1


def _run_bq(xyzt, nxyz):
    return pl.pallas_call(
        _bq_body,
        grid=(_B, _NBQ),
        in_specs=[
            pl.BlockSpec((1, 3, _N1), lambda b, c: (b, 0, 0)),
            pl.BlockSpec((_SBLK, 3), lambda b, c: (b * _NBQ + c, 0)),
        ],
        out_specs=pl.BlockSpec((_SBLK, _NS), lambda b, c: (b * _NBQ + c, 0)),
        out_shape=jax.ShapeDtypeStruct((_B * _NP, _NS), jnp.int32),
        compiler_params=pltpu.CompilerParams(
            dimension_semantics=("parallel", "parallel")),
    )(xyzt, nxyz)


def _sc_gather(table, idx):
    n, D = idx.shape[0], table.shape[1]
    info = plsc.get_sparse_core_info()
    nw = info.num_cores * info.num_subcores
    b_per_w = n // nw
    nchunk = 2
    chunk = b_per_w // nchunk
    mesh = plsc.VectorSubcoreMesh(core_axis_name="c", subcore_axis_name="s")

    @functools.partial(
        pl.kernel, mesh=mesh,
        out_type=jax.ShapeDtypeStruct((n, D), jnp.float32),
        scratch_types=[
            pltpu.VMEM((chunk,), jnp.int32),
            pltpu.VMEM((chunk, D), jnp.float32),
            pltpu.SemaphoreType.DMA,
        ],
    )
    def k(table_hbm, idx_hbm, out_hbm, idx_v, rows_v, sem):
        wid = lax.axis_index("s") * info.num_cores + lax.axis_index("c")
        for j in range(nchunk):
            base = wid * b_per_w + j * chunk
            pltpu.sync_copy(idx_hbm.at[pl.ds(base, chunk)], idx_v)
            pltpu.async_copy(table_hbm.at[idx_v], rows_v, sem).wait()
            pltpu.sync_copy(rows_v, out_hbm.at[pl.ds(base, chunk)])

    return k(table, idx)


def _ln(h, g, b):
    m = jnp.mean(h, axis=-1, keepdims=True)
    v = jnp.mean((h - m) ** 2, axis=-1, keepdims=True)
    return (h - m) / jnp.sqrt(v + _EPS) * g + b


def _dense_body(g_ref, nrep_ref, xT_ref, xyzT_ref,
                Wq_ref, bq_ref, Wk_ref, bk_ref, Wv_ref, bv_ref,
                pA_ref, pAb_ref, plg_ref, plb_ref, pB_ref, pBb_ref,
                qA_ref, qAb_ref, qlg_ref, qlb_ref, qB_ref, qBb_ref,
                l1g_ref, l1b_ref, wA_ref, wAb_ref,
                l2g_ref, l2b_ref, wB_ref, wBb_ref,
                wlog_ref, xv_ref):
    g = g_ref[0]
    gx1 = g[:, 0:_IN_P]
    gxyz = g[:, _IN_P:_IN_P + 3]
    grel = gxyz - nrep_ref[0]

    x_k = jnp.dot(gx1, Wk_ref[...],
                  preferred_element_type=jnp.float32) + bk_ref[...]
    x_v = jnp.dot(gx1, Wv_ref[...],
                  preferred_element_type=jnp.float32) + bv_ref[...]

    h = jnp.dot(grel, pA_ref[...],
                preferred_element_type=jnp.float32) + pAb_ref[...]
    h = jax.nn.relu(_ln(h, plg_ref[...], plb_ref[...]))
    p_r = jnp.dot(h, pB_ref[...],
                  preferred_element_type=jnp.float32) + pBb_ref[...]
    pr_red = p_r[:, 0:_MID] + p_r[:, _MID:_OUT_P]

    a = xyzT_ref[0]
    h1 = jnp.dot(a, qA_ref[...],
                 preferred_element_type=jnp.float32) + qAb_ref[...]
    h1 = jax.nn.relu(_ln(h1, qlg_ref[...], qlb_ref[...]))
    p_r_1 = jnp.dot(h1, qB_ref[...],
                    preferred_element_type=jnp.float32) + qBb_ref[...]
    x_q = jnp.dot(xT_ref[0], Wq_ref[...],
                  preferred_element_type=jnp.float32) + bq_ref[...]
    q = x_q + p_r_1
    q_rep = jnp.broadcast_to(q[:, None, :], (_CB, _NS, _MID)).reshape(
        _CB * _NS, _MID)

    r_qk = x_k - q_rep + pr_red
    hw = jax.nn.relu(_ln(r_qk, l1g_ref[...], l1b_ref[...]))
    hw = jnp.dot(hw, wA_ref[...],
                 preferred_element_type=jnp.float32) + wAb_ref[...]
    hw = jax.nn.relu(_ln(hw, l2g_ref[...], l2b_ref[...]))
    wlog = jnp.dot(hw, wB_ref[...],
                   preferred_element_type=jnp.float32) + wBb_ref[...]

    wlog_ref[0] = wlog
    xv_ref[0] = x_v + p_r


def _run_dense(g, nrep, xT, xyzT, weights):
    nblk = _B * _NCB
    wspecs = [pl.BlockSpec(w.shape, functools.partial(
        lambda nd, b, c: (0,) * nd, w.ndim)) for w in weights]
    return pl.pallas_call(
        _dense_body,
        grid=(_B, _NCB),
        in_specs=[
            pl.BlockSpec((1, _CB * _NS, 128),
                         lambda b, c: (b * _NCB + c, 0, 0)),
            pl.BlockSpec((1, _CB * _NS, 3),
                         lambda b, c: (b * _NCB + c, 0, 0)),
            pl.BlockSpec((1, _CB, _IN_P), lambda b, c: (b * _NCB + c, 0, 0)),
            pl.BlockSpec((1, _CB, 3), lambda b, c: (b * _NCB + c, 0, 0)),
        ] + wspecs,
        out_specs=[
            pl.BlockSpec((1, _CB * _NS, _NS),
                         lambda b, c: (b * _NCB + c, 0, 0)),
            pl.BlockSpec((1, _CB * _NS, _OUT_P),
                         lambda b, c: (b * _NCB + c, 0, 0)),
        ],
        out_shape=[
            jax.ShapeDtypeStruct((nblk, _CB * _NS, _NS), jnp.float32),
            jax.ShapeDtypeStruct((nblk, _CB * _NS, _OUT_P), jnp.float32),
        ],
        compiler_params=pltpu.CompilerParams(
            dimension_semantics=("parallel", "parallel")),
    )(g, nrep, xT, xyzT, *weights)


def _agg_body(wlog_ref, xv_ref, out_ref):
    wlog = wlog_ref[0]
    mx = jnp.max(wlog, axis=0, keepdims=True)
    e = jnp.exp(wlog - mx)
    w = e / jnp.sum(e, axis=0, keepdims=True)
    acc = jnp.zeros((_NP, _OUT_P), jnp.float32)
    for t in range(_NS):
        wt = w[:, t * _NS:(t + 1) * _NS]
        wt8 = jnp.concatenate([wt] * _SHARE, axis=1)
        acc = acc + xv_ref[0, :, t, :] * wt8
    out_ref[0] = acc


def _run_agg(wlog, xv):
    return pl.pallas_call(
        _agg_body,
        grid=(_B,),
        in_specs=[
            pl.BlockSpec((1, _NP, _NS * _NS), lambda b: (b, 0, 0)),
            pl.BlockSpec((1, _NP, _NS, _OUT_P), lambda b: (b, 0, 0, 0)),
        ],
        out_specs=pl.BlockSpec((1, _NP, _OUT_P), lambda b: (b, 0, 0)),
        out_shape=jax.ShapeDtypeStruct((_B, _NP, _OUT_P), jnp.float32),
        compiler_params=pltpu.CompilerParams(
            dimension_semantics=("parallel",)),
    )(wlog, xv)


def kernel(x, xyz, x1, xyz1, Wq, bq, Wk, bk, Wv, bv, pA_W, pA_b, p_ln_g,
           p_ln_b, pB_W, pB_b, qA_W, qA_b, q_ln_g, q_ln_b, qB_W, qB_b,
           w_ln1_g, w_ln1_b, wA_W, wA_b, w_ln2_g, w_ln2_b, wB_W, wB_b):
    xyzt = jnp.transpose(xyz1, (0, 2, 1))
    xyzr = xyzt.reshape(_B, 3, 64, 128)
    fps_flat, nxyz_flat = _run_fps(xyzr)
    new_xyz = nxyz_flat.reshape(_B, _NP, 3)

    idx_g = _run_bq(xyzt, nxyz_flat)

    table = jnp.concatenate(
        [x1, xyz1, jnp.zeros((_B, _N1, 61), jnp.float32)],
        axis=-1).reshape(_B * _N1, 128)
    g = _sc_gather(table, idx_g.reshape(-1))

    nrep = jnp.broadcast_to(
        new_xyz[:, :, None, :], (_B, _NP, _NS, 3))
    g4 = g.reshape(_B * _NCB, _CB * _NS, 128)
    nrep4 = nrep.reshape(_B * _NCB, _CB * _NS, 3)
    xT = jnp.transpose(x, (0, 2, 1)).reshape(_B * _NCB, _CB, _IN_P)
    xyzT = jnp.transpose(xyz, (0, 2, 1)).reshape(_B * _NCB, _CB, 3)

    weights = [
        Wq, bq.reshape(1, -1), Wk, bk.reshape(1, -1), Wv, bv.reshape(1, -1),
        pA_W, pA_b.reshape(1, -1), p_ln_g.reshape(1, -1),
        p_ln_b.reshape(1, -1), pB_W, pB_b.reshape(1, -1),
        qA_W, qA_b.reshape(1, -1), q_ln_g.reshape(1, -1),
        q_ln_b.reshape(1, -1), qB_W, qB_b.reshape(1, -1),
        w_ln1_g.reshape(1, -1), w_ln1_b.reshape(1, -1), wA_W,
        wA_b.reshape(1, -1), w_ln2_g.reshape(1, -1), w_ln2_b.reshape(1, -1),
        wB_W, wB_b.reshape(1, -1),
    ]
    wlog, xv = _run_dense(g4, nrep4, xT, xyzT, weights)

    wlog2 = wlog.reshape(_B, _NP, _NS * _NS)
    xv4 = xv.reshape(_B, _NP, _NS, _OUT_P)
    out = _run_agg(wlog2, xv4)
    return out, new_xyz


# speedup vs baseline: 6.9006x; 1.1410x over previous
import functools

import jax
import jax.numpy as jnp
from jax import lax
from jax.experimental import pallas as pl
from jax.experimental.pallas import tpu as pltpu
from jax.experimental.pallas import tpu_sc as plsc

_B, _N1, _NP, _NS = 4, 8192, 512, 16
_IN_P, _OUT_P, _MID, _SHARE = 64, 128, 64, 8
_R2 = 0.2 ** 2
_EPS = 1e-5
_HI = jax.lax.Precision.HIGHEST

_SBLK = 64
_CB = 128
_NBQ = _NP // _SBLK
_NCB = _NP // _CB


def _fps_body(xyz_ref, fps_ref, nxyz_ref):
    XYZ = [[xyz_ref[b, c] for c in range(3)] for b in range(_B)]
    I = (lax.broadcasted_iota(jnp.int32, (64, 128), 0) * 128
         + lax.broadcasted_iota(jnp.int32, (64, 128), 1))
    lane = lax.broadcasted_iota(jnp.int32, (1, 128), 1)

    def step(k, carry):
        Ds, fs = carry
        newDs, newfs = [], []
        for b in range(_B):
            D, f = Ds[b], fs[b]
            X, Y, Z = XYZ[b]
            row = f // 128
            col = f - row * 128
            sel = lane == col
            xr = xyz_ref[b, 0, pl.ds(row, 1), :]
            yr = xyz_ref[b, 1, pl.ds(row, 1), :]
            zr = xyz_ref[b, 2, pl.ds(row, 1), :]
            cx = jnp.sum(jnp.where(sel, xr, 0.0))
            cy = jnp.sum(jnp.where(sel, yr, 0.0))
            cz = jnp.sum(jnp.where(sel, zr, 0.0))
            dist = (X - cx) ** 2 + (Y - cy) ** 2 + (Z - cz) ** 2
            D = jnp.minimum(D, dist)
            m = jnp.max(D)
            f_next = jnp.min(jnp.where(D == m, I, jnp.int32(_N1)))
            fps_ref[pl.ds(b * _NP + k, 1), :] = jnp.full((1, 1), f, jnp.int32)
            nxyz_ref[pl.ds(b * _NP + k, 1), 0:1] = jnp.full((1, 1), cx)
            nxyz_ref[pl.ds(b * _NP + k, 1), 1:2] = jnp.full((1, 1), cy)
            nxyz_ref[pl.ds(b * _NP + k, 1), 2:3] = jnp.full((1, 1), cz)
            newDs.append(D)
            newfs.append(f_next)
        return tuple(newDs), tuple(newfs)

    D0 = jnp.full((64, 128), 1e10, jnp.float32)
    lax.fori_loop(0, _NP, step,
                  ((D0,) * _B, (jnp.int32(0),) * _B))


def _run_fps(xyzr):
    return pl.pallas_call(
        _fps_body,
        in_specs=[pl.BlockSpec((_B, 3, 64, 128), lambda: (0, 0, 0, 0))],
        out_specs=[
            pl.BlockSpec((_B * _NP, 1), lambda: (0, 0)),
            pl.BlockSpec((_B * _NP, 3), lambda: (0, 0)),
        ],
        out_shape=[
            jax.ShapeDtypeStruct((_B * _NP, 1), jnp.int32),
            jax.ShapeDtypeStruct((_B * _NP, 3), jnp.float32),
        ],
    )(xyzr)


def _bq_body(xyzt_ref, nxyz_ref, idx_ref):
    b = pl.program_id(0)
    px = xyzt_ref[0, 0:1, :]
    py = xyzt_ref[0, 1:2, :]
    pz = xyzt_ref[0, 2:3, :]
    pn = px * px + py * py + pz * pz
    nx = nxyz_ref[:, 0:1]
    ny = nxyz_ref[:, 1:2]
    nz = nxyz_ref[:, 2:3]
    cn = nx * nx + ny * ny + nz * nz
    dot = jnp.dot(nxyz_ref[...], xyzt_ref[0],
                  preferred_element_type=jnp.float32)
    sqd = cn + pn - 2.0 * dot
    iotaL = lax.broadcasted_iota(jnp.int32, (_SBLK, _N1), 1)
    cand0 = jnp.where(sqd <= _R2, iotaL, jnp.int32(_N1))
    slot = lax.broadcasted_iota(jnp.int32, (_SBLK, _NS), 1)

    def step(t, carry):
        cand, acc = carry
        m = jnp.min(cand, axis=1, keepdims=True)
        acc = jnp.where(slot == t, m, acc)
        cand = jnp.where(cand == m, jnp.int32(_N1), cand)
        return cand, acc

    _, acc = lax.fori_loop(
        0, _NS, step, (cand0, jnp.zeros((_SBLK, _NS), jnp.int32)))
    first = acc[:, 0:1]
    acc = jnp.where(acc == _N1, first, acc)
    idx_ref[...] = acc + b * _N1


def _run_bq(xyzt, nxyz):
    return pl.pallas_call(
        _bq_body,
        grid=(_B, _NBQ),
        in_specs=[
            pl.BlockSpec((1, 3, _N1), lambda b, c: (b, 0, 0)),
            pl.BlockSpec((_SBLK, 3), lambda b, c: (b * _NBQ + c, 0)),
        ],
        out_specs=pl.BlockSpec((_SBLK, _NS), lambda b, c: (b * _NBQ + c, 0)),
        out_shape=jax.ShapeDtypeStruct((_B * _NP, _NS), jnp.int32),
        compiler_params=pltpu.CompilerParams(
            dimension_semantics=("parallel", "parallel")),
    )(xyzt, nxyz)


def _sc_gather(table, idx):
    n, D = idx.shape[0], table.shape[1]
    info = plsc.get_sparse_core_info()
    nw = info.num_cores * info.num_subcores
    b_per_w = n // nw
    nchunk = 2
    chunk = b_per_w // nchunk
    mesh = plsc.VectorSubcoreMesh(core_axis_name="c", subcore_axis_name="s")

    @functools.partial(
        pl.kernel, mesh=mesh,
        out_type=jax.ShapeDtypeStruct((n, D), jnp.float32),
        scratch_types=[
            pltpu.VMEM((chunk,), jnp.int32),
            pltpu.VMEM((chunk, D), jnp.float32),
            pltpu.SemaphoreType.DMA,
        ],
    )
    def k(table_hbm, idx_hbm, out_hbm, idx_v, rows_v, sem):
        wid = lax.axis_index("s") * info.num_cores + lax.axis_index("c")
        for j in range(nchunk):
            base = wid * b_per_w + j * chunk
            pltpu.sync_copy(idx_hbm.at[pl.ds(base, chunk)], idx_v)
            pltpu.async_copy(table_hbm.at[idx_v], rows_v, sem).wait()
            pltpu.sync_copy(rows_v, out_hbm.at[pl.ds(base, chunk)])

    return k(table, idx)


def _ln(h, g, b):
    m = jnp.mean(h, axis=-1, keepdims=True)
    v = jnp.mean((h - m) ** 2, axis=-1, keepdims=True)
    return (h - m) / jnp.sqrt(v + _EPS) * g + b


def _dense_body(g_ref, nrep_ref, xT_ref, xyzT_ref,
                Wq_ref, bq_ref, Wk_ref, bk_ref, Wv_ref, bv_ref,
                pA_ref, pAb_ref, plg_ref, plb_ref, pB_ref, pBb_ref,
                qA_ref, qAb_ref, qlg_ref, qlb_ref, qB_ref, qBb_ref,
                l1g_ref, l1b_ref, wA_ref, wAb_ref,
                l2g_ref, l2b_ref, wB_ref, wBb_ref,
                wlog_ref, xv_ref):
    g = g_ref[0]
    gx1 = g[:, 0:_IN_P]
    gxyz = g[:, _IN_P:_IN_P + 3]
    grel = gxyz - nrep_ref[0]

    x_k = jnp.dot(gx1, Wk_ref[...],
                  preferred_element_type=jnp.float32) + bk_ref[...]
    x_v = jnp.dot(gx1, Wv_ref[...],
                  preferred_element_type=jnp.float32) + bv_ref[...]

    h = jnp.dot(grel, pA_ref[...],
                preferred_element_type=jnp.float32) + pAb_ref[...]
    h = jax.nn.relu(_ln(h, plg_ref[...], plb_ref[...]))
    p_r = jnp.dot(h, pB_ref[...],
                  preferred_element_type=jnp.float32) + pBb_ref[...]
    pr_red = p_r[:, 0:_MID] + p_r[:, _MID:_OUT_P]

    a = xyzT_ref[0]
    h1 = jnp.dot(a, qA_ref[...],
                 preferred_element_type=jnp.float32) + qAb_ref[...]
    h1 = jax.nn.relu(_ln(h1, qlg_ref[...], qlb_ref[...]))
    p_r_1 = jnp.dot(h1, qB_ref[...],
                    preferred_element_type=jnp.float32) + qBb_ref[...]
    x_q = jnp.dot(xT_ref[0], Wq_ref[...],
                  preferred_element_type=jnp.float32) + bq_ref[...]
    q = x_q + p_r_1
    q_rep = jnp.broadcast_to(q[:, None, :], (_CB, _NS, _MID)).reshape(
        _CB * _NS, _MID)

    r_qk = x_k - q_rep + pr_red
    hw = jax.nn.relu(_ln(r_qk, l1g_ref[...], l1b_ref[...]))
    hw = jnp.dot(hw, wA_ref[...],
                 preferred_element_type=jnp.float32) + wAb_ref[...]
    hw = jax.nn.relu(_ln(hw, l2g_ref[...], l2b_ref[...]))
    wlog = jnp.dot(hw, wB_ref[...],
                   preferred_element_type=jnp.float32) + wBb_ref[...]

    wlog_ref[0] = wlog
    xv_ref[0] = x_v + p_r


def _run_dense(g, nrep, xT, xyzT, weights):
    nblk = _B * _NCB
    wspecs = [pl.BlockSpec(w.shape, functools.partial(
        lambda nd, b, c: (0,) * nd, w.ndim)) for w in weights]
    return pl.pallas_call(
        _dense_body,
        grid=(_B, _NCB),
        in_specs=[
            pl.BlockSpec((1, _CB * _NS, 128),
                         lambda b, c: (b * _NCB + c, 0, 0)),
            pl.BlockSpec((1, _CB * _NS, 3),
                         lambda b, c: (b * _NCB + c, 0, 0)),
            pl.BlockSpec((1, _CB, _IN_P), lambda b, c: (b * _NCB + c, 0, 0)),
            pl.BlockSpec((1, _CB, 3), lambda b, c: (b * _NCB + c, 0, 0)),
        ] + wspecs,
        out_specs=[
            pl.BlockSpec((1, _CB * _NS, _NS),
                         lambda b, c: (b * _NCB + c, 0, 0)),
            pl.BlockSpec((1, _CB * _NS, _OUT_P),
                         lambda b, c: (b * _NCB + c, 0, 0)),
        ],
        out_shape=[
            jax.ShapeDtypeStruct((nblk, _CB * _NS, _NS), jnp.float32),
            jax.ShapeDtypeStruct((nblk, _CB * _NS, _OUT_P), jnp.float32),
        ],
        compiler_params=pltpu.CompilerParams(
            dimension_semantics=("parallel", "parallel")),
    )(g, nrep, xT, xyzT, *weights)


def _agg_body(wlog_ref, xv_ref, out_ref):
    wlog = wlog_ref[0]
    mx = jnp.max(wlog, axis=0, keepdims=True)
    e = jnp.exp(wlog - mx)
    w = e / jnp.sum(e, axis=0, keepdims=True)
    acc = jnp.zeros((_NP, _OUT_P), jnp.float32)
    for t in range(_NS):
        wt = w[:, t * _NS:(t + 1) * _NS]
        wt8 = jnp.concatenate([wt] * _SHARE, axis=1)
        acc = acc + xv_ref[0, :, t, :] * wt8
    out_ref[0] = acc


def _run_agg(wlog, xv):
    return pl.pallas_call(
        _agg_body,
        grid=(_B,),
        in_specs=[
            pl.BlockSpec((1, _NP, _NS * _NS), lambda b: (b, 0, 0)),
            pl.BlockSpec((1, _NP, _NS, _OUT_P), lambda b: (b, 0, 0, 0)),
        ],
        out_specs=pl.BlockSpec((1, _NP, _OUT_P), lambda b: (b, 0, 0)),
        out_shape=jax.ShapeDtypeStruct((_B, _NP, _OUT_P), jnp.float32),
        compiler_params=pltpu.CompilerParams(
            dimension_semantics=("parallel",)),
    )(wlog, xv)


def kernel(x, xyz, x1, xyz1, Wq, bq, Wk, bk, Wv, bv, pA_W, pA_b, p_ln_g,
           p_ln_b, pB_W, pB_b, qA_W, qA_b, q_ln_g, q_ln_b, qB_W, qB_b,
           w_ln1_g, w_ln1_b, wA_W, wA_b, w_ln2_g, w_ln2_b, wB_W, wB_b):
    xyzt = jnp.transpose(xyz1, (0, 2, 1))
    xyzr = xyzt.reshape(_B, 3, 64, 128)
    fps_flat, nxyz_flat = _run_fps(xyzr)
    new_xyz = nxyz_flat.reshape(_B, _NP, 3)

    idx_g = _run_bq(xyzt, nxyz_flat)

    table = jnp.concatenate(
        [x1, xyz1, jnp.zeros((_B, _N1, 61), jnp.float32)],
        axis=-1).reshape(_B * _N1, 128)
    g = _sc_gather(table, idx_g.reshape(-1))

    nrep = jnp.broadcast_to(
        new_xyz[:, :, None, :], (_B, _NP, _NS, 3))
    g4 = g.reshape(_B * _NCB, _CB * _NS, 128)
    nrep4 = nrep.reshape(_B * _NCB, _CB * _NS, 3)
    xT = jnp.transpose(x, (0, 2, 1)).reshape(_B * _NCB, _CB, _IN_P)
    xyzT = jnp.transpose(xyz, (0, 2, 1)).reshape(_B * _NCB, _CB, 3)

    weights = [
        Wq, bq.reshape(1, -1), Wk, bk.reshape(1, -1), Wv, bv.reshape(1, -1),
        pA_W, pA_b.reshape(1, -1), p_ln_g.reshape(1, -1),
        p_ln_b.reshape(1, -1), pB_W, pB_b.reshape(1, -1),
        qA_W, qA_b.reshape(1, -1), q_ln_g.reshape(1, -1),
        q_ln_b.reshape(1, -1), qB_W, qB_b.reshape(1, -1),
        w_ln1_g.reshape(1, -1), w_ln1_b.reshape(1, -1), wA_W,
        wA_b.reshape(1, -1), w_ln2_g.reshape(1, -1), w_ln2_b.reshape(1, -1),
        wB_W, wB_b.reshape(1, -1),
    ]
    wlog, xv = _run_dense(g4, nrep4, xT, xyzT, weights)

    wlog2 = wlog.reshape(_B, _NP, _NS * _NS)
    xv4 = xv.reshape(_B, _NP, _NS, _OUT_P)
    out = _run_agg(wlog2, xv4)
    return out, new_xyz


# speedup vs baseline: 6.9016x; 1.0001x over previous
import functools

import jax
import jax.numpy as jnp
from jax import lax
from jax.experimental import pallas as pl
from jax.experimental.pallas import tpu as pltpu
from jax.experimental.pallas import tpu_sc as plsc

_B, _N1, _NP, _NS = 4, 8192, 512, 16
_IN_P, _OUT_P, _MID, _SHARE = 64, 128, 64, 8
_R2 = 0.2 ** 2
_EPS = 1e-5

_SBLK = 64
_CB = 128
_NBQ = _NP // _SBLK
_NCB = _NP // _CB


def _fps_body(xyz_ref, fps_ref, nxyz_ref):
    XYZ = [[xyz_ref[b, c] for c in range(3)] for b in range(_B)]
    I = (lax.broadcasted_iota(jnp.int32, (64, 128), 0) * 128
         + lax.broadcasted_iota(jnp.int32, (64, 128), 1))
    lane = lax.broadcasted_iota(jnp.int32, (1, 128), 1)

    def step(k, carry):
        Ds, fs = carry
        newDs, newfs = [], []
        for b in range(_B):
            D, f = Ds[b], fs[b]
            X, Y, Z = XYZ[b]
            row = f // 128
            col = f - row * 128
            sel = lane == col
            xr = xyz_ref[b, 0, pl.ds(row, 1), :]
            yr = xyz_ref[b, 1, pl.ds(row, 1), :]
            zr = xyz_ref[b, 2, pl.ds(row, 1), :]
            cx = jnp.sum(jnp.where(sel, xr, 0.0))
            cy = jnp.sum(jnp.where(sel, yr, 0.0))
            cz = jnp.sum(jnp.where(sel, zr, 0.0))
            dist = (X - cx) ** 2 + (Y - cy) ** 2 + (Z - cz) ** 2
            D = jnp.minimum(D, dist)
            m = jnp.max(D)
            f_next = jnp.min(jnp.where(D == m, I, jnp.int32(_N1)))
            fps_ref[pl.ds(b * _NP + k, 1), :] = jnp.full((1, 1), f, jnp.int32)
            nxyz_ref[pl.ds(b * _NP + k, 1), 0:1] = jnp.full((1, 1), cx)
            nxyz_ref[pl.ds(b * _NP + k, 1), 1:2] = jnp.full((1, 1), cy)
            nxyz_ref[pl.ds(b * _NP + k, 1), 2:3] = jnp.full((1, 1), cz)
            newDs.append(D)
            newfs.append(f_next)
        return tuple(newDs), tuple(newfs)

    D0 = jnp.full((64, 128), 1e10, jnp.float32)
    lax.fori_loop(0, _NP, step,
                  ((D0,) * _B, (jnp.int32(0),) * _B))


def _run_fps(xyzr):
    return pl.pallas_call(
        _fps_body,
        in_specs=[pl.BlockSpec((_B, 3, 64, 128), lambda: (0, 0, 0, 0))],
        out_specs=[
            pl.BlockSpec((_B * _NP, 1), lambda: (0, 0)),
            pl.BlockSpec((_B * _NP, 3), lambda: (0, 0)),
        ],
        out_shape=[
            jax.ShapeDtypeStruct((_B * _NP, 1), jnp.int32),
            jax.ShapeDtypeStruct((_B * _NP, 3), jnp.float32),
        ],
    )(xyzr)


def _bq_body(xyzt_ref, nxyz_ref, idx_ref):
    b = pl.program_id(0)
    px = xyzt_ref[0, 0:1, :]
    py = xyzt_ref[0, 1:2, :]
    pz = xyzt_ref[0, 2:3, :]
    pn = px * px + py * py + pz * pz
    nx = nxyz_ref[:, 0:1]
    ny = nxyz_ref[:, 1:2]
    nz = nxyz_ref[:, 2:3]
    cn = nx * nx + ny * ny + nz * nz
    dot = jnp.dot(nxyz_ref[...], xyzt_ref[0],
                  preferred_element_type=jnp.float32)
    sqd = cn + pn - 2.0 * dot
    iotaL = lax.broadcasted_iota(jnp.int32, (_SBLK, _N1), 1)
    cand0 = jnp.where(sqd <= _R2, iotaL, jnp.int32(_N1))
    slot = lax.broadcasted_iota(jnp.int32, (_SBLK, _NS), 1)

    def step(t, carry):
        cand, acc = carry
        m = jnp.min(cand, axis=1, keepdims=True)
        acc = jnp.where(slot == t, m, acc)
        cand = jnp.where(cand == m, jnp.int32(_N1), cand)
        return cand, acc

    _, acc = lax.fori_loop(
        0, _NS, step, (cand0, jnp.zeros((_SBLK, _NS), jnp.int32)))
    first = acc[:, 0:1]
    acc = jnp.where(acc == _N1, first, acc)
    idx_ref[...] = acc + b * _N1


def _run_bq(xyzt, nxyz):
    return pl.pallas_call(
        _bq_body,
        grid=(_B, _NBQ),
        in_specs=[
            pl.BlockSpec((1, 3, _N1), lambda b, c: (b, 0, 0)),
            pl.BlockSpec((_SBLK, 3), lambda b, c: (b * _NBQ + c, 0)),
        ],
        out_specs=pl.BlockSpec((_SBLK, _NS), lambda b, c: (b * _NBQ + c, 0)),
        out_shape=jax.ShapeDtypeStruct((_B * _NP, _NS), jnp.int32),
        compiler_params=pltpu.CompilerParams(
            dimension_semantics=("parallel", "parallel")),
    )(xyzt, nxyz)


def _sc_gather(table, idx):
    n, D = idx.shape[0], table.shape[1]
    info = plsc.get_sparse_core_info()
    nw = info.num_cores * info.num_subcores
    b_per_w = n // nw
    nchunk = 2
    chunk = b_per_w // nchunk
    mesh = plsc.VectorSubcoreMesh(core_axis_name="c", subcore_axis_name="s")

    @functools.partial(
        pl.kernel, mesh=mesh,
        out_type=jax.ShapeDtypeStruct((n, D), jnp.float32),
        scratch_types=[
            pltpu.VMEM((chunk,), jnp.int32),
            pltpu.VMEM((chunk, D), jnp.float32),
            pltpu.SemaphoreType.DMA,
        ],
    )
    def k(table_hbm, idx_hbm, out_hbm, idx_v, rows_v, sem):
        wid = lax.axis_index("s") * info.num_cores + lax.axis_index("c")
        for j in range(nchunk):
            base = wid * b_per_w + j * chunk
            pltpu.sync_copy(idx_hbm.at[pl.ds(base, chunk)], idx_v)
            pltpu.async_copy(table_hbm.at[idx_v], rows_v, sem).wait()
            pltpu.sync_copy(rows_v, out_hbm.at[pl.ds(base, chunk)])

    return k(table, idx)


def _ln(h, g, b):
    m = jnp.mean(h, axis=-1, keepdims=True)
    v = jnp.mean((h - m) ** 2, axis=-1, keepdims=True)
    return (h - m) / jnp.sqrt(v + _EPS) * g + b


def _dense_body(g_ref, nrep_ref, xT_ref, xyzT_ref,
                Wq_ref, bq_ref, Wk_ref, bk_ref, Wv_ref, bv_ref,
                pA_ref, pAb_ref, plg_ref, plb_ref, pB_ref, pBb_ref,
                qA_ref, qAb_ref, qlg_ref, qlb_ref, qB_ref, qBb_ref,
                l1g_ref, l1b_ref, wA_ref, wAb_ref,
                l2g_ref, l2b_ref, wB_ref, wBb_ref,
                wlog_ref, xv_ref):
    g = g_ref[0]
    gx1 = g[:, 0:_IN_P]
    gxyz = g[:, _IN_P:_IN_P + 3]
    grel = gxyz - nrep_ref[0]

    x_k = jnp.dot(gx1, Wk_ref[...],
                  preferred_element_type=jnp.float32) + bk_ref[...]
    x_v = jnp.dot(gx1, Wv_ref[...],
                  preferred_element_type=jnp.float32) + bv_ref[...]

    h = jnp.dot(grel, pA_ref[...],
                preferred_element_type=jnp.float32) + pAb_ref[...]
    h = jax.nn.relu(_ln(h, plg_ref[...], plb_ref[...]))
    p_r = jnp.dot(h, pB_ref[...],
                  preferred_element_type=jnp.float32) + pBb_ref[...]
    pr_red = p_r[:, 0:_MID] + p_r[:, _MID:_OUT_P]

    a = xyzT_ref[0]
    h1 = jnp.dot(a, qA_ref[...],
                 preferred_element_type=jnp.float32) + qAb_ref[...]
    h1 = jax.nn.relu(_ln(h1, qlg_ref[...], qlb_ref[...]))
    p_r_1 = jnp.dot(h1, qB_ref[...],
                    preferred_element_type=jnp.float32) + qBb_ref[...]
    x_q = jnp.dot(xT_ref[0], Wq_ref[...],
                  preferred_element_type=jnp.float32) + bq_ref[...]
    q = x_q + p_r_1
    q_rep = jnp.broadcast_to(q[:, None, :], (_CB, _NS, _MID)).reshape(
        _CB * _NS, _MID)

    r_qk = x_k - q_rep + pr_red
    hw = jax.nn.relu(_ln(r_qk, l1g_ref[...], l1b_ref[...]))
    hw = jnp.dot(hw, wA_ref[...],
                 preferred_element_type=jnp.float32) + wAb_ref[...]
    hw = jax.nn.relu(_ln(hw, l2g_ref[...], l2b_ref[...]))
    wlog = jnp.dot(hw, wB_ref[...],
                   preferred_element_type=jnp.float32) + wBb_ref[...]

    wlog_ref[0] = wlog
    xv_ref[0] = x_v + p_r


def _run_dense(g, nrep, xT, xyzT, weights):
    nblk = _B * _NCB
    wspecs = [pl.BlockSpec(w.shape, functools.partial(
        lambda nd, b, c: (0,) * nd, w.ndim)) for w in weights]
    return pl.pallas_call(
        _dense_body,
        grid=(_B, _NCB),
        in_specs=[
            pl.BlockSpec((1, _CB * _NS, 128),
                         lambda b, c: (b * _NCB + c, 0, 0)),
            pl.BlockSpec((1, _CB * _NS, 3),
                         lambda b, c: (b * _NCB + c, 0, 0)),
            pl.BlockSpec((1, _CB, _IN_P), lambda b, c: (b * _NCB + c, 0, 0)),
            pl.BlockSpec((1, _CB, 3), lambda b, c: (b * _NCB + c, 0, 0)),
        ] + wspecs,
        out_specs=[
            pl.BlockSpec((1, _CB * _NS, _NS),
                         lambda b, c: (b * _NCB + c, 0, 0)),
            pl.BlockSpec((1, _CB * _NS, _OUT_P),
                         lambda b, c: (b * _NCB + c, 0, 0)),
        ],
        out_shape=[
            jax.ShapeDtypeStruct((nblk, _CB * _NS, _NS), jnp.float32),
            jax.ShapeDtypeStruct((nblk, _CB * _NS, _OUT_P), jnp.float32),
        ],
        compiler_params=pltpu.CompilerParams(
            dimension_semantics=("parallel", "parallel")),
    )(g, nrep, xT, xyzT, *weights)


def _agg_body(wlog_ref, xv_ref, out_ref):
    wlog = wlog_ref[0]
    mx = jnp.max(wlog, axis=0, keepdims=True)
    e = jnp.exp(wlog - mx)
    w = e / jnp.sum(e, axis=0, keepdims=True)
    acc = jnp.zeros((_NP, _OUT_P), jnp.float32)
    for t in range(_NS):
        wt = w[:, t * _NS:(t + 1) * _NS]
        wt8 = jnp.concatenate([wt] * _SHARE, axis=1)
        acc = acc + xv_ref[0, :, t, :] * wt8
    out_ref[0] = acc


def _run_agg(wlog, xv):
    return pl.pallas_call(
        _agg_body,
        grid=(_B,),
        in_specs=[
            pl.BlockSpec((1, _NP, _NS * _NS), lambda b: (b, 0, 0)),
            pl.BlockSpec((1, _NP, _NS, _OUT_P), lambda b: (b, 0, 0, 0)),
        ],
        out_specs=pl.BlockSpec((1, _NP, _OUT_P), lambda b: (b, 0, 0)),
        out_shape=jax.ShapeDtypeStruct((_B, _NP, _OUT_P), jnp.float32),
        compiler_params=pltpu.CompilerParams(
            dimension_semantics=("parallel",)),
    )(wlog, xv)


def kernel(x, xyz, x1, xyz1, Wq, bq, Wk, bk, Wv, bv, pA_W, pA_b, p_ln_g,
           p_ln_b, pB_W, pB_b, qA_W, qA_b, q_ln_g, q_ln_b, qB_W, qB_b,
           w_ln1_g, w_ln1_b, wA_W, wA_b, w_ln2_g, w_ln2_b, wB_W, wB_b):
    xyzt = jnp.transpose(xyz1, (0, 2, 1))
    xyzr = xyzt.reshape(_B, 3, 64, 128)
    fps_flat, nxyz_flat = _run_fps(xyzr)
    new_xyz = nxyz_flat.reshape(_B, _NP, 3)

    idx_g = _run_bq(xyzt, nxyz_flat)

    table = jnp.concatenate(
        [x1, xyz1, jnp.zeros((_B, _N1, 61), jnp.float32)],
        axis=-1).reshape(_B * _N1, 128)
    g = _sc_gather(table, idx_g.reshape(-1))

    nrep = jnp.broadcast_to(
        new_xyz[:, :, None, :], (_B, _NP, _NS, 3))
    g4 = g.reshape(_B * _NCB, _CB * _NS, 128)
    nrep4 = nrep.reshape(_B * _NCB, _CB * _NS, 3)
    xT = jnp.transpose(x, (0, 2, 1)).reshape(_B * _NCB, _CB, _IN_P)
    xyzT = jnp.transpose(xyz, (0, 2, 1)).reshape(_B * _NCB, _CB, 3)

    weights = [
        Wq, bq.reshape(1, -1), Wk, bk.reshape(1, -1), Wv, bv.reshape(1, -1),
        pA_W, pA_b.reshape(1, -1), p_ln_g.reshape(1, -1),
        p_ln_b.reshape(1, -1), pB_W, pB_b.reshape(1, -1),
        qA_W, qA_b.reshape(1, -1), q_ln_g.reshape(1, -1),
        q_ln_b.reshape(1, -1), qB_W, qB_b.reshape(1, -1),
        w_ln1_g.reshape(1, -1), w_ln1_b.reshape(1, -1), wA_W,
        wA_b.reshape(1, -1), w_ln2_g.reshape(1, -1), w_ln2_b.reshape(1, -1),
        wB_W, wB_b.reshape(1, -1),
    ]
    wlog, xv = _run_dense(g4, nrep4, xT, xyzT, weights)

    wlog2 = wlog.reshape(_B, _NP, _NS * _NS)
    xv4 = xv.reshape(_B, _NP, _NS, _OUT_P)
    out = _run_agg(wlog2, xv4)
    return out, new_xyz
